# baseline (speedup 1.0000x reference)
import functools

import jax
import jax.numpy as jnp
from jax import lax
from jax.experimental import pallas as pl
from jax.experimental.pallas import tpu as pltpu
from jax.experimental.pallas import tpu_sc as plsc

N = 10000
E = 320000
DIN = 128
HID = 128
HEADS = 8
NG = 32

NCORES = 2
NSUB = 16
EPT = E // NSUB
SB = 2000
NSBS = EPT // SB


def _sc_edge_pass(H, C, TPR):
    WID = H * C + 128
    HALF = N // NCORES
    NWIN = -(-HALF // TPR)
    NR = -(-NWIN // NSUB)

    mesh = plsc.VectorSubcoreMesh(
        core_axis_name="c", subcore_axis_name="s",
        num_cores=NCORES, num_subcores=NSUB)

    @functools.partial(
        pl.kernel, mesh=mesh,
        compiler_params=pltpu.CompilerParams(needs_layout_passes=False),
        out_type=jax.ShapeDtypeStruct((N, WID), jnp.float32),
        scratch_types=[
            pltpu.VMEM((TPR, WID), jnp.float32),
            pltpu.VMEM((SB,), jnp.int32),
            pltpu.VMEM((SB,), jnp.int32),
            pltpu.VMEM((SB,), jnp.int32),
            pltpu.VMEM((SB,), jnp.int32),
            pltpu.VMEM((SB + 16,), jnp.int32),
            pltpu.VMEM((SB + 16,), jnp.int32),
            pltpu.VMEM((16, WID), jnp.float32),
            pltpu.VMEM((16, 128), jnp.float32),
            pltpu.SemaphoreType.DMA,
            pltpu.SemaphoreType.DMA,
        ],
    )
    def body(src_hbm, dst_hbm, featA, featB, out_hbm,
             acc, srcbuf0, dstbuf0, srcbuf1, dstbuf1, csrc, cdst,
             gbuf, bbuf, sem0, sem1):
        cid = lax.axis_index("c")
        sid = lax.axis_index("s")
        lanes = lax.iota(jnp.int32, 16)
        cbase = cid * HALF
        cend = cbase + HALF
        zero16 = jnp.zeros((16,), jnp.float32)

        def round_body(r, _):
            wlo = cbase + (r * NSUB + sid) * TPR
            whi = jnp.minimum(wlo + TPR, cend)

            @pl.when(wlo < cend)
            def _():
                def zrow(l, _):
                    for ccg in range(WID // 16):
                        acc[l, pl.ds(ccg * 16, 16)] = zero16
                    return 0
                lax.fori_loop(0, TPR, zrow, 0)

                wlov = jnp.broadcast_to(wlo, (16,))
                whiv = jnp.broadcast_to(whi, (16,))

                def scan_proc(srcbuf, dstbuf):
                    def scan_g(g, cnt):
                        sv = srcbuf[pl.ds(g * 16, 16)]
                        dv = dstbuf[pl.ds(g * 16, 16)]
                        m = (dv >= wlov) & (dv < whiv)

                        def hitcase(c):
                            mi = jnp.where(m, jnp.full((16,), 1, jnp.int32),
                                           jnp.full((16,), 0, jnp.int32))
                            pos = (plsc.cumsum(mi) - mi
                                   + jnp.broadcast_to(c, (16,)))
                            plsc.store_scatter(csrc, [pos], sv, mask=m)
                            plsc.store_scatter(cdst, [pos], dv, mask=m)
                            return pos[15] + mi[15]

                        return lax.cond(jnp.any(m), hitcase, lambda c: c, cnt)
                    cnt = lax.fori_loop(0, SB // 16, scan_g, jnp.int32(0))

                    ng = (cnt + 15) // 16

                    def proc(j, _):
                        validv = (j * 16 + lanes) < cnt
                        sv = csrc[pl.ds(j * 16, 16)]
                        dv = cdst[pl.ds(j * 16, 16)]
                        sv = jnp.where(validv, sv, 0)
                        dlv = jnp.where(validv, dv - wlo, 0)
                        dv = jnp.where(validv, dv, 0)
                        pltpu.sync_copy(featA.at[sv], gbuf)
                        pltpu.sync_copy(featB.at[dv], bbuf)

                        for l in range(16):
                            e = (gbuf[l, pl.ds(H * C, 16)]
                                 + bbuf[l, pl.ds(0, 16)])
                            e = jnp.where(e >= 0, e, e * jnp.float32(0.2))
                            w = jnp.exp(e)
                            ok = (j * 16 + l) < cnt
                            w = jnp.where((lanes < H) & ok, w,
                                          jnp.float32(0.0))
                            dl = dlv[l]
                            plsc.addupdate(acc.at[dl, pl.ds(H * C, 16)], w)
                            for k in range(H):
                                ws = w[k]
                                for ccg in range(C // 16):
                                    off = k * C + ccg * 16
                                    plsc.addupdate(
                                        acc.at[dl, pl.ds(off, 16)],
                                        gbuf[l, pl.ds(off, 16)] * ws)
                        return 0
                    lax.fori_loop(0, ng, proc, 0)

                pltpu.async_copy(src_hbm.at[pl.ds(0, SB)], srcbuf0, sem0)
                pltpu.async_copy(dst_hbm.at[pl.ds(0, SB)], dstbuf0, sem0)

                def pair_body(t, _):
                    b0 = (2 * t) * SB
                    b1 = (2 * t + 1) * SB
                    b2 = (2 * t + 2) * SB
                    pltpu.make_async_copy(
                        src_hbm.at[pl.ds(b0, SB)], srcbuf0, sem0).wait()
                    pltpu.make_async_copy(
                        dst_hbm.at[pl.ds(b0, SB)], dstbuf0, sem0).wait()
                    pltpu.async_copy(
                        src_hbm.at[pl.ds(b1, SB)], srcbuf1, sem1)
                    pltpu.async_copy(
                        dst_hbm.at[pl.ds(b1, SB)], dstbuf1, sem1)
                    scan_proc(srcbuf0, dstbuf0)
                    pltpu.make_async_copy(
                        src_hbm.at[pl.ds(b1, SB)], srcbuf1, sem1).wait()
                    pltpu.make_async_copy(
                        dst_hbm.at[pl.ds(b1, SB)], dstbuf1, sem1).wait()

                    @pl.when(b2 < E)
                    def _():
                        pltpu.async_copy(
                            src_hbm.at[pl.ds(b2, SB)], srcbuf0, sem0)
                        pltpu.async_copy(
                            dst_hbm.at[pl.ds(b2, SB)], dstbuf0, sem0)
                    scan_proc(srcbuf1, dstbuf1)
                    return 0
                lax.fori_loop(0, E // SB // 2, pair_body, 0)

                for t in range(TPR // 16):
                    rs = t * 16

                    @pl.when(wlo + rs + 16 <= whi)
                    def _():
                        pltpu.sync_copy(acc.at[pl.ds(rs, 16)],
                                        out_hbm.at[pl.ds(wlo + rs, 16)])

                    @pl.when((wlo + rs + 16 > whi) & (wlo + rs + 8 <= whi))
                    def _():
                        pltpu.sync_copy(acc.at[pl.ds(rs, 8)],
                                        out_hbm.at[pl.ds(wlo + rs, 8)])
            return 0
        lax.fori_loop(0, NR, round_body, 0)

    return body


_sc_edge_l1 = _sc_edge_pass(HEADS, HID, 80)
_sc_edge_l2 = _sc_edge_pass(1, HID, 336)


BR = 1000


def _tc1_body(x_ref, wcat_ref, wb_ref, oa_ref, ob_ref):
    x = x_ref[...]
    oa_ref[...] = jnp.dot(x, wcat_ref[...], preferred_element_type=jnp.float32)
    ob_ref[...] = jnp.dot(x, wb_ref[...], preferred_element_type=jnp.float32)


def _tc1(x, wcat, wb):
    wid = wcat.shape[1]
    return pl.pallas_call(
        _tc1_body,
        grid=(N // BR,),
        in_specs=[
            pl.BlockSpec((BR, DIN), lambda i: (i, 0)),
            pl.BlockSpec((DIN, wid), lambda i: (0, 0)),
            pl.BlockSpec((DIN, 128), lambda i: (0, 0)),
        ],
        out_specs=[
            pl.BlockSpec((BR, wid), lambda i: (i, 0)),
            pl.BlockSpec((BR, 128), lambda i: (i, 0)),
        ],
        out_shape=[
            jax.ShapeDtypeStruct((N, wid), jnp.float32),
            jax.ShapeDtypeStruct((N, 128), jnp.float32),
        ],
    )(x, wcat, wb)


def _elu(v):
    return jnp.where(v > 0, v, jnp.exp(jnp.minimum(v, 0.0)) - 1.0)


def _tc2_body(acc_ref, fa_ref, fb_ref, b_ref, wcat_ref, wb_ref,
              oa_ref, ob_ref, z_ref):
    H, C = HEADS, HID
    acc = acc_ref[...]
    asrc = fa_ref[:, H * C:H * C + H]
    adst = fb_ref[:, 0:H]
    s = asrc + adst
    wself = jnp.exp(jnp.where(s >= 0, s, s * 0.2))
    den = acc[:, H * C:H * C + H] + wself + 1e-16
    for k in range(H):
        sl = slice(k * C, (k + 1) * C)
        z = (acc[:, sl] + wself[:, k:k + 1] * fa_ref[:, sl]) / den[:, k:k + 1]
        z_ref[:, sl] = _elu(z + b_ref[0:1, sl])
    zc = z_ref[...]
    oa_ref[...] = jnp.dot(zc, wcat_ref[...], preferred_element_type=jnp.float32)
    ob_ref[...] = jnp.dot(zc, wb_ref[...], preferred_element_type=jnp.float32)


def _tc2(acc1, fa1, fb1, b1r, w2cat, w2b):
    wid1 = fa1.shape[1]
    wid2 = w2cat.shape[1]
    return pl.pallas_call(
        _tc2_body,
        grid=(N // BR,),
        in_specs=[
            pl.BlockSpec((BR, wid1), lambda i: (i, 0)),
            pl.BlockSpec((BR, wid1), lambda i: (i, 0)),
            pl.BlockSpec((BR, 128), lambda i: (i, 0)),
            pl.BlockSpec((1, HEADS * HID), lambda i: (0, 0)),
            pl.BlockSpec((HEADS * HID, wid2), lambda i: (0, 0)),
            pl.BlockSpec((HEADS * HID, 128), lambda i: (0, 0)),
        ],
        out_specs=[
            pl.BlockSpec((BR, wid2), lambda i: (i, 0)),
            pl.BlockSpec((BR, 128), lambda i: (i, 0)),
        ],
        out_shape=[
            jax.ShapeDtypeStruct((N, wid2), jnp.float32),
            jax.ShapeDtypeStruct((N, 128), jnp.float32),
        ],
        scratch_shapes=[pltpu.VMEM((BR, HEADS * HID), jnp.float32)],
    )(acc1, fa1, fb1, b1r, w2cat, w2b)


def _tc3_body(acc_ref, fa_ref, fb_ref, b_ref, batch_ref, wout_ref, bout_ref,
              out_ref, pooled_ref, cnt_ref):
    i = pl.program_id(0)

    @pl.when(i == 0)
    def _():
        pooled_ref[...] = jnp.zeros_like(pooled_ref)
        cnt_ref[...] = jnp.zeros_like(cnt_ref)

    C = HID
    acc = acc_ref[...]
    asrc = fa_ref[:, C:C + 1]
    adst = fb_ref[:, 0:1]
    s = asrc + adst
    wself = jnp.exp(jnp.where(s >= 0, s, s * 0.2))
    den = acc[:, C:C + 1] + wself + 1e-16
    z = (acc[:, 0:C] + wself * fa_ref[:, 0:C]) / den
    y = _elu(z + b_ref[0:1, :])

    b2d = batch_ref[0]
    ohT = (lax.broadcasted_iota(jnp.int32, (NG, BR), 0) == b2d
           ).astype(jnp.float32)
    pooled_ref[...] += jnp.dot(ohT, y, preferred_element_type=jnp.float32)
    cnt_ref[...] += jnp.dot(ohT, jnp.ones((BR, HID), jnp.float32),
                            preferred_element_type=jnp.float32)

    @pl.when(i == N // BR - 1)
    def _():
        p = pooled_ref[...] / jnp.maximum(cnt_ref[...], 1.0)
        r = jnp.dot(p, wout_ref[...], preferred_element_type=jnp.float32)
        out_ref[...] = jax.nn.sigmoid(r + bout_ref[...])


def _tc3(acc2, fa2, fb2, b2r, batch3, wout, boutr):
    wid2 = fa2.shape[1]
    return pl.pallas_call(
        _tc3_body,
        grid=(N // BR,),
        in_specs=[
            pl.BlockSpec((BR, wid2), lambda i: (i, 0)),
            pl.BlockSpec((BR, wid2), lambda i: (i, 0)),
            pl.BlockSpec((BR, 128), lambda i: (i, 0)),
            pl.BlockSpec((1, HID), lambda i: (0, 0)),
            pl.BlockSpec((1, 1, BR), lambda i: (i, 0, 0)),
            pl.BlockSpec((HID, 1), lambda i: (0, 0)),
            pl.BlockSpec((1, 1), lambda i: (0, 0)),
        ],
        out_specs=pl.BlockSpec((NG, 1), lambda i: (0, 0)),
        out_shape=jax.ShapeDtypeStruct((NG, 1), jnp.float32),
        scratch_shapes=[
            pltpu.VMEM((NG, HID), jnp.float32),
            pltpu.VMEM((NG, HID), jnp.float32),
        ],
    )(acc2, fa2, fb2, b2r, batch3, wout, boutr)


def _block_diag_logit_mat(a):
    H, C = a.shape
    eye = jnp.eye(H, dtype=a.dtype)
    return (a[:, :, None] * eye[:, None, :]).reshape(H * C, H)


def kernel(x, edge_index, batch, W1, a_src1, a_dst1, b1, W2, a_src2,
           a_dst2, b2, Wout, bout):
    src = edge_index[0].astype(jnp.int32)
    dst = edge_index[1].astype(jnp.int32)

    H, C = HEADS, HID
    w1a = W1 @ _block_diag_logit_mat(a_src1)
    w1b = W1 @ _block_diag_logit_mat(a_dst1)
    zpad = jnp.zeros((DIN, 120), jnp.float32)
    w1cat = jnp.concatenate([W1, w1a, zpad], axis=1)
    w1bp = jnp.concatenate([w1b, zpad], axis=1)

    w2a = W2 @ a_src2.T
    w2b = W2 @ a_dst2.T
    zpad2 = jnp.zeros((H * C, 127), jnp.float32)
    w2cat = jnp.concatenate([W2, w2a, zpad2], axis=1)
    w2bp = jnp.concatenate([w2b, zpad2], axis=1)

    b1r = b1.reshape(1, H * C)
    b2r = b2.reshape(1, HID)
    boutr = bout.reshape(1, 1)
    batch3 = batch.astype(jnp.int32).reshape(N // BR, 1, BR)

    fa1, fb1 = _tc1(x, w1cat, w1bp)
    acc1 = _sc_edge_l1(src, dst, fa1, fb1)
    fa2, fb2 = _tc2(acc1, fa1, fb1, b1r, w2cat, w2bp)
    acc2 = _sc_edge_l2(src, dst, fa2, fb2)
    return _tc3(acc2, fa2, fb2, b2r, batch3, Wout, boutr)


# speedup vs baseline: 1.0153x; 1.0153x over previous
import functools

import jax
import jax.numpy as jnp
from jax import lax
from jax.experimental import pallas as pl
from jax.experimental.pallas import tpu as pltpu
from jax.experimental.pallas import tpu_sc as plsc

N = 10000
E = 320000
DIN = 128
HID = 128
HEADS = 8
NG = 32

NCORES = 2
NSUB = 16
EPT = E // NSUB
SB = 2000
NSBS = EPT // SB


def _sc_edge_pass(H, C, TPR):
    WID = H * C + 128
    HALF = N // NCORES
    NWIN = -(-HALF // TPR)
    NR = -(-NWIN // NSUB)

    mesh = plsc.VectorSubcoreMesh(
        core_axis_name="c", subcore_axis_name="s",
        num_cores=NCORES, num_subcores=NSUB)

    @functools.partial(
        pl.kernel, mesh=mesh,
        compiler_params=pltpu.CompilerParams(needs_layout_passes=False),
        out_type=jax.ShapeDtypeStruct((N, WID), jnp.float32),
        scratch_types=[
            pltpu.VMEM((TPR, WID), jnp.float32),
            pltpu.VMEM((SB,), jnp.int32),
            pltpu.VMEM((SB,), jnp.int32),
            pltpu.VMEM((SB,), jnp.int32),
            pltpu.VMEM((SB,), jnp.int32),
            pltpu.VMEM((SB + 16,), jnp.int32),
            pltpu.VMEM((SB + 16,), jnp.int32),
            pltpu.VMEM((16, WID), jnp.float32),
            pltpu.VMEM((16, 128), jnp.float32),
            pltpu.SemaphoreType.DMA,
            pltpu.SemaphoreType.DMA,
        ],
    )
    def body(src_hbm, dst_hbm, featA, featB, out_hbm,
             acc, srcbuf0, dstbuf0, srcbuf1, dstbuf1, csrc, cdst,
             gbuf, bbuf, sem0, sem1):
        cid = lax.axis_index("c")
        sid = lax.axis_index("s")
        lanes = lax.iota(jnp.int32, 16)
        cbase = cid * HALF
        cend = cbase + HALF
        zero16 = jnp.zeros((16,), jnp.float32)

        def round_body(r, _):
            wlo = cbase + (r * NSUB + sid) * TPR
            whi = jnp.minimum(wlo + TPR, cend)

            @pl.when(wlo < cend)
            def _():
                def zrow(l, _):
                    for ccg in range(WID // 16):
                        acc[l, pl.ds(ccg * 16, 16)] = zero16
                    return 0
                lax.fori_loop(0, TPR, zrow, 0)

                wlov = jnp.broadcast_to(wlo, (16,))
                whiv = jnp.broadcast_to(whi, (16,))

                def scan_proc(srcbuf, dstbuf):
                    def scan_g(g, cnt):
                        dv = dstbuf[pl.ds(g * 16, 16)]
                        m = (dv >= wlov) & (dv < whiv)
                        nhit = plsc.all_reduce_population_count(m)[0]

                        def hitcase(c):
                            sv = srcbuf[pl.ds(g * 16, 16)]
                            mi = jnp.where(m, jnp.full((16,), 1, jnp.int32),
                                           jnp.full((16,), 0, jnp.int32))
                            pos = (plsc.cumsum(mi) - mi
                                   + jnp.broadcast_to(c, (16,)))
                            plsc.store_scatter(csrc, [pos], sv, mask=m)
                            plsc.store_scatter(cdst, [pos], dv, mask=m)
                            return c + nhit

                        return lax.cond(nhit > 0, hitcase, lambda c: c, cnt)
                    cnt = lax.fori_loop(0, SB // 16, scan_g, jnp.int32(0))

                    ng = (cnt + 15) // 16

                    def proc(j, _):
                        validv = (j * 16 + lanes) < cnt
                        sv = csrc[pl.ds(j * 16, 16)]
                        dv = cdst[pl.ds(j * 16, 16)]
                        sv = jnp.where(validv, sv, 0)
                        dlv = jnp.where(validv, dv - wlo, 0)
                        dv = jnp.where(validv, dv, 0)
                        pltpu.sync_copy(featA.at[sv], gbuf)
                        pltpu.sync_copy(featB.at[dv], bbuf)

                        for l in range(16):
                            e = (gbuf[l, pl.ds(H * C, 16)]
                                 + bbuf[l, pl.ds(0, 16)])
                            e = jnp.where(e >= 0, e, e * jnp.float32(0.2))
                            w = jnp.exp(e)
                            ok = (j * 16 + l) < cnt
                            w = jnp.where((lanes < H) & ok, w,
                                          jnp.float32(0.0))
                            dl = dlv[l]
                            plsc.addupdate(acc.at[dl, pl.ds(H * C, 16)], w)
                            for k in range(H):
                                ws = w[k]
                                for ccg in range(C // 16):
                                    off = k * C + ccg * 16
                                    plsc.addupdate(
                                        acc.at[dl, pl.ds(off, 16)],
                                        gbuf[l, pl.ds(off, 16)] * ws)
                        return 0
                    lax.fori_loop(0, ng, proc, 0)

                pltpu.async_copy(src_hbm.at[pl.ds(0, SB)], srcbuf0, sem0)
                pltpu.async_copy(dst_hbm.at[pl.ds(0, SB)], dstbuf0, sem0)

                def pair_body(t, _):
                    b0 = (2 * t) * SB
                    b1 = (2 * t + 1) * SB
                    b2 = (2 * t + 2) * SB
                    pltpu.make_async_copy(
                        src_hbm.at[pl.ds(b0, SB)], srcbuf0, sem0).wait()
                    pltpu.make_async_copy(
                        dst_hbm.at[pl.ds(b0, SB)], dstbuf0, sem0).wait()
                    pltpu.async_copy(
                        src_hbm.at[pl.ds(b1, SB)], srcbuf1, sem1)
                    pltpu.async_copy(
                        dst_hbm.at[pl.ds(b1, SB)], dstbuf1, sem1)
                    scan_proc(srcbuf0, dstbuf0)
                    pltpu.make_async_copy(
                        src_hbm.at[pl.ds(b1, SB)], srcbuf1, sem1).wait()
                    pltpu.make_async_copy(
                        dst_hbm.at[pl.ds(b1, SB)], dstbuf1, sem1).wait()

                    @pl.when(b2 < E)
                    def _():
                        pltpu.async_copy(
                            src_hbm.at[pl.ds(b2, SB)], srcbuf0, sem0)
                        pltpu.async_copy(
                            dst_hbm.at[pl.ds(b2, SB)], dstbuf0, sem0)
                    scan_proc(srcbuf1, dstbuf1)
                    return 0
                lax.fori_loop(0, E // SB // 2, pair_body, 0)

                for t in range(TPR // 16):
                    rs = t * 16

                    @pl.when(wlo + rs + 16 <= whi)
                    def _():
                        pltpu.sync_copy(acc.at[pl.ds(rs, 16)],
                                        out_hbm.at[pl.ds(wlo + rs, 16)])

                    @pl.when((wlo + rs + 16 > whi) & (wlo + rs + 8 <= whi))
                    def _():
                        pltpu.sync_copy(acc.at[pl.ds(rs, 8)],
                                        out_hbm.at[pl.ds(wlo + rs, 8)])
            return 0
        lax.fori_loop(0, NR, round_body, 0)

    return body


_sc_edge_l1 = _sc_edge_pass(HEADS, HID, 80)
_sc_edge_l2 = _sc_edge_pass(1, HID, 336)


BR = 1000


def _tc1_body(x_ref, wcat_ref, wb_ref, oa_ref, ob_ref):
    x = x_ref[...]
    oa_ref[...] = jnp.dot(x, wcat_ref[...], preferred_element_type=jnp.float32)
    ob_ref[...] = jnp.dot(x, wb_ref[...], preferred_element_type=jnp.float32)


def _tc1(x, wcat, wb):
    wid = wcat.shape[1]
    return pl.pallas_call(
        _tc1_body,
        grid=(N // BR,),
        in_specs=[
            pl.BlockSpec((BR, DIN), lambda i: (i, 0)),
            pl.BlockSpec((DIN, wid), lambda i: (0, 0)),
            pl.BlockSpec((DIN, 128), lambda i: (0, 0)),
        ],
        out_specs=[
            pl.BlockSpec((BR, wid), lambda i: (i, 0)),
            pl.BlockSpec((BR, 128), lambda i: (i, 0)),
        ],
        out_shape=[
            jax.ShapeDtypeStruct((N, wid), jnp.float32),
            jax.ShapeDtypeStruct((N, 128), jnp.float32),
        ],
    )(x, wcat, wb)


def _elu(v):
    return jnp.where(v > 0, v, jnp.exp(jnp.minimum(v, 0.0)) - 1.0)


def _tc2_body(acc_ref, fa_ref, fb_ref, b_ref, wcat_ref, wb_ref,
              oa_ref, ob_ref, z_ref):
    H, C = HEADS, HID
    acc = acc_ref[...]
    asrc = fa_ref[:, H * C:H * C + H]
    adst = fb_ref[:, 0:H]
    s = asrc + adst
    wself = jnp.exp(jnp.where(s >= 0, s, s * 0.2))
    den = acc[:, H * C:H * C + H] + wself + 1e-16
    for k in range(H):
        sl = slice(k * C, (k + 1) * C)
        z = (acc[:, sl] + wself[:, k:k + 1] * fa_ref[:, sl]) / den[:, k:k + 1]
        z_ref[:, sl] = _elu(z + b_ref[0:1, sl])
    zc = z_ref[...]
    oa_ref[...] = jnp.dot(zc, wcat_ref[...], preferred_element_type=jnp.float32)
    ob_ref[...] = jnp.dot(zc, wb_ref[...], preferred_element_type=jnp.float32)


def _tc2(acc1, fa1, fb1, b1r, w2cat, w2b):
    wid1 = fa1.shape[1]
    wid2 = w2cat.shape[1]
    return pl.pallas_call(
        _tc2_body,
        grid=(N // BR,),
        in_specs=[
            pl.BlockSpec((BR, wid1), lambda i: (i, 0)),
            pl.BlockSpec((BR, wid1), lambda i: (i, 0)),
            pl.BlockSpec((BR, 128), lambda i: (i, 0)),
            pl.BlockSpec((1, HEADS * HID), lambda i: (0, 0)),
            pl.BlockSpec((HEADS * HID, wid2), lambda i: (0, 0)),
            pl.BlockSpec((HEADS * HID, 128), lambda i: (0, 0)),
        ],
        out_specs=[
            pl.BlockSpec((BR, wid2), lambda i: (i, 0)),
            pl.BlockSpec((BR, 128), lambda i: (i, 0)),
        ],
        out_shape=[
            jax.ShapeDtypeStruct((N, wid2), jnp.float32),
            jax.ShapeDtypeStruct((N, 128), jnp.float32),
        ],
        scratch_shapes=[pltpu.VMEM((BR, HEADS * HID), jnp.float32)],
    )(acc1, fa1, fb1, b1r, w2cat, w2b)


def _tc3_body(acc_ref, fa_ref, fb_ref, b_ref, batch_ref, wout_ref, bout_ref,
              out_ref, pooled_ref, cnt_ref):
    i = pl.program_id(0)

    @pl.when(i == 0)
    def _():
        pooled_ref[...] = jnp.zeros_like(pooled_ref)
        cnt_ref[...] = jnp.zeros_like(cnt_ref)

    C = HID
    acc = acc_ref[...]
    asrc = fa_ref[:, C:C + 1]
    adst = fb_ref[:, 0:1]
    s = asrc + adst
    wself = jnp.exp(jnp.where(s >= 0, s, s * 0.2))
    den = acc[:, C:C + 1] + wself + 1e-16
    z = (acc[:, 0:C] + wself * fa_ref[:, 0:C]) / den
    y = _elu(z + b_ref[0:1, :])

    b2d = batch_ref[0]
    ohT = (lax.broadcasted_iota(jnp.int32, (NG, BR), 0) == b2d
           ).astype(jnp.float32)
    pooled_ref[...] += jnp.dot(ohT, y, preferred_element_type=jnp.float32)
    cnt_ref[...] += jnp.dot(ohT, jnp.ones((BR, HID), jnp.float32),
                            preferred_element_type=jnp.float32)

    @pl.when(i == N // BR - 1)
    def _():
        p = pooled_ref[...] / jnp.maximum(cnt_ref[...], 1.0)
        r = jnp.dot(p, wout_ref[...], preferred_element_type=jnp.float32)
        out_ref[...] = jax.nn.sigmoid(r + bout_ref[...])


def _tc3(acc2, fa2, fb2, b2r, batch3, wout, boutr):
    wid2 = fa2.shape[1]
    return pl.pallas_call(
        _tc3_body,
        grid=(N // BR,),
        in_specs=[
            pl.BlockSpec((BR, wid2), lambda i: (i, 0)),
            pl.BlockSpec((BR, wid2), lambda i: (i, 0)),
            pl.BlockSpec((BR, 128), lambda i: (i, 0)),
            pl.BlockSpec((1, HID), lambda i: (0, 0)),
            pl.BlockSpec((1, 1, BR), lambda i: (i, 0, 0)),
            pl.BlockSpec((HID, 1), lambda i: (0, 0)),
            pl.BlockSpec((1, 1), lambda i: (0, 0)),
        ],
        out_specs=pl.BlockSpec((NG, 1), lambda i: (0, 0)),
        out_shape=jax.ShapeDtypeStruct((NG, 1), jnp.float32),
        scratch_shapes=[
            pltpu.VMEM((NG, HID), jnp.float32),
            pltpu.VMEM((NG, HID), jnp.float32),
        ],
    )(acc2, fa2, fb2, b2r, batch3, wout, boutr)


def _block_diag_logit_mat(a):
    H, C = a.shape
    eye = jnp.eye(H, dtype=a.dtype)
    return (a[:, :, None] * eye[:, None, :]).reshape(H * C, H)


def kernel(x, edge_index, batch, W1, a_src1, a_dst1, b1, W2, a_src2,
           a_dst2, b2, Wout, bout):
    src = edge_index[0].astype(jnp.int32)
    dst = edge_index[1].astype(jnp.int32)

    H, C = HEADS, HID
    w1a = W1 @ _block_diag_logit_mat(a_src1)
    w1b = W1 @ _block_diag_logit_mat(a_dst1)
    zpad = jnp.zeros((DIN, 120), jnp.float32)
    w1cat = jnp.concatenate([W1, w1a, zpad], axis=1)
    w1bp = jnp.concatenate([w1b, zpad], axis=1)

    w2a = W2 @ a_src2.T
    w2b = W2 @ a_dst2.T
    zpad2 = jnp.zeros((H * C, 127), jnp.float32)
    w2cat = jnp.concatenate([W2, w2a, zpad2], axis=1)
    w2bp = jnp.concatenate([w2b, zpad2], axis=1)

    b1r = b1.reshape(1, H * C)
    b2r = b2.reshape(1, HID)
    boutr = bout.reshape(1, 1)
    batch3 = batch.astype(jnp.int32).reshape(N // BR, 1, BR)

    fa1, fb1 = _tc1(x, w1cat, w1bp)
    acc1 = _sc_edge_l1(src, dst, fa1, fb1)
    fa2, fb2 = _tc2(acc1, fa1, fb1, b1r, w2cat, w2bp)
    acc2 = _sc_edge_l2(src, dst, fa2, fb2)
    return _tc3(acc2, fa2, fb2, b2r, batch3, Wout, boutr)
